# Initial kernel scaffold; baseline (speedup 1.0000x reference)
import jax
import jax.numpy as jnp
from jax.experimental import pallas as pl


def kernel(x, edge_index, batch, conv1_W, conv1_b, conv2_W, conv2_b, fc1_W, fc1_b, fc2_W, fc2_b):
    raise NotImplementedError("write your pallas kernel here")


# speedup vs baseline: 23.1123x; 23.1123x over previous
import functools

import jax
import jax.numpy as jnp
from jax import lax
from jax.experimental import pallas as pl
from jax.experimental.pallas import tpu as pltpu
from jax.experimental.pallas import tpu_sc as plsc

NC = 2
NS = 16
NW = NC * NS
LANES = 128

G = 32
LATENT = 64


def _mesh():
  return plsc.VectorSubcoreMesh(
      core_axis_name="c", subcore_axis_name="s", num_cores=NC,
      num_subcores=NS)


def _make_deg(n_pad, k):
  rs = n_pad // NS

  @functools.partial(
      pl.kernel,
      mesh=_mesh(),
      compiler_params=pltpu.CompilerParams(use_tc_tiling_on_sc=False),
      out_type=jax.ShapeDtypeStruct((NC, n_pad), jnp.float32),
      scratch_types=[
          pltpu.VMEM((k, LANES), jnp.int32),
          pltpu.VMEM((LANES,), jnp.float32),
          pltpu.VMEM_SHARED((n_pad,), jnp.float32),
      ],
  )
  def deg_kernel(dst_hbm, out_hbm, idx_v, ones_v, accum):
    c = lax.axis_index("c")
    s = lax.axis_index("s")
    wid = c * NS + s
    for i in range(LANES // 16):
      ones_v[pl.ds(i * 16, 16)] = jnp.ones((16,), jnp.float32)
    for j in range(rs // LANES):
      pltpu.sync_copy(ones_v, accum.at[pl.ds(s * rs + j * LANES, LANES)])
    pltpu.sync_copy(dst_hbm.at[wid], idx_v)
    plsc.subcore_barrier()

    def body(j, carry):
      pltpu.sync_copy(ones_v, accum.at[idx_v.at[j]], add=True)
      return carry

    lax.fori_loop(0, k, body, 0)
    plsc.subcore_barrier()
    pltpu.sync_copy(accum.at[pl.ds(s * rs, rs)],
                    out_hbm.at[c, pl.ds(s * rs, rs)])

  return deg_kernel


def _make_agg(n_pad, k, f):
  rs = n_pad // NS

  @functools.partial(
      pl.kernel,
      mesh=_mesh(),
      compiler_params=pltpu.CompilerParams(use_tc_tiling_on_sc=False),
      out_type=jax.ShapeDtypeStruct((NC, n_pad, f), jnp.float32),
      scratch_types=[
          pltpu.VMEM((k, LANES), jnp.int32),
          pltpu.VMEM((k, LANES), jnp.int32),
          pltpu.VMEM((LANES, f), jnp.float32),
          pltpu.VMEM_SHARED((n_pad, f), jnp.float32),
          pltpu.SemaphoreType.DMA,
      ],
  )
  def agg_kernel(src_hbm, dst_hbm, msg_hbm, out_hbm, src_v, dst_v, rows_v,
                 accum, sem):
    c = lax.axis_index("c")
    s = lax.axis_index("s")
    wid = c * NS + s
    pltpu.sync_copy(msg_hbm.at[pl.ds(s * rs, rs), :],
                    accum.at[pl.ds(s * rs, rs), :])
    pltpu.sync_copy(src_hbm.at[wid], src_v)
    pltpu.sync_copy(dst_hbm.at[wid], dst_v)
    plsc.subcore_barrier()

    def body(j, carry):
      pltpu.async_copy(msg_hbm.at[src_v.at[j]], rows_v, sem).wait()
      pltpu.sync_copy(rows_v, accum.at[dst_v.at[j]], add=True)
      return carry

    lax.fori_loop(0, k, body, 0)
    plsc.subcore_barrier()
    pltpu.sync_copy(accum.at[pl.ds(s * rs, rs), :],
                    out_hbm.at[c, pl.ds(s * rs, rs), :])

  return agg_kernel


def _dinv_block(d0b, d1b):
  deg = d0b + d1b - 1.0
  return lax.rsqrt(jnp.maximum(deg, 1.0))


def _msg1_body(x_ref, w_ref, d0_ref, d1_ref, out_ref):
  xw = jnp.dot(x_ref[...], w_ref[...], preferred_element_type=jnp.float32)
  out_ref[...] = xw * _dinv_block(d0_ref[...], d1_ref[...])


def _msg2_body(p0_ref, p1_ref, m1_ref, d0_ref, d1_ref, b1_ref, w2_ref,
               out_ref):
  dinv = _dinv_block(d0_ref[...], d1_ref[...])
  h1 = dinv * (p0_ref[...] + p1_ref[...] - m1_ref[...]) + b1_ref[...]
  h1 = jnp.maximum(h1, 0.0)
  xw2 = jnp.dot(h1, w2_ref[...], preferred_element_type=jnp.float32)
  out_ref[...] = xw2 * dinv


def _final_body(q0_ref, q1_ref, m2_ref, d0_ref, d1_ref, b2_ref, batch_ref,
                fc1w_ref, fc1b_ref, fc2w_ref, fc2b_ref, out_ref):
  n_pad = q0_ref.shape[0]
  dinv = _dinv_block(d0_ref[...], d1_ref[...])
  h2 = dinv * (q0_ref[...] + q1_ref[...] - m2_ref[...]) + b2_ref[...]
  b_flat = batch_ref[...]
  gids = lax.broadcasted_iota(jnp.int32, (G, n_pad), 0)
  seg = jnp.where(gids == b_flat, 1.0, 0.0)
  sums = jnp.dot(seg, h2, preferred_element_type=jnp.float32)
  cnt = jnp.sum(seg, axis=1, keepdims=True)
  pooled = sums / jnp.maximum(cnt, 1.0)
  z = jnp.dot(pooled, fc1w_ref[...], preferred_element_type=jnp.float32)
  z = jnp.maximum(z + fc1b_ref[...], 0.0)
  out_ref[...] = (
      jnp.dot(z, fc2w_ref[...], preferred_element_type=jnp.float32)
      + fc2b_ref[...])


def kernel(x, edge_index, batch, conv1_W, conv1_b, conv2_W, conv2_b,
           fc1_W, fc1_b, fc2_W, fc2_b):
  n, f_in = x.shape
  e = edge_index.shape[1]
  f1 = conv1_W.shape[1]
  f2 = conv2_W.shape[1]

  blk = 1024
  n_pad = ((n + 1 + blk - 1) // blk) * blk
  per_tile = -(-e // NW)
  k = -(-per_tile // LANES)
  e_pad = NW * k * LANES

  x_pad = jnp.concatenate(
      [x, jnp.zeros((n_pad - n, f_in), jnp.float32)], axis=0)
  pad_idx = jnp.full((e_pad - e,), n, jnp.int32)
  src = jnp.concatenate([edge_index[0].astype(jnp.int32), pad_idx])
  dst = jnp.concatenate([edge_index[1].astype(jnp.int32), pad_idx])
  src_t = src.reshape(NW, k, LANES)
  dst_t = dst.reshape(NW, k, LANES)
  batch_pad = jnp.concatenate(
      [batch.astype(jnp.int32), jnp.full((n_pad - n,), G, jnp.int32)])
  batch_r = batch_pad.reshape(1, n_pad)

  d_part = _make_deg(n_pad, k)(dst_t)
  d0r = d_part[0].reshape(n_pad, 1)
  d1r = d_part[1].reshape(n_pad, 1)

  grid = n_pad // blk
  dspec = pl.BlockSpec((blk, 1), lambda i: (i, 0))

  msg1 = pl.pallas_call(
      _msg1_body,
      grid=(grid,),
      in_specs=[
          pl.BlockSpec((blk, f_in), lambda i: (i, 0)),
          pl.BlockSpec((f_in, f1), lambda i: (0, 0)),
          dspec, dspec,
      ],
      out_specs=pl.BlockSpec((blk, f1), lambda i: (i, 0)),
      out_shape=jax.ShapeDtypeStruct((n_pad, f1), jnp.float32),
  )(x_pad, conv1_W, d0r, d1r)

  p = _make_agg(n_pad, k, f1)(src_t, dst_t, msg1)

  msg2 = pl.pallas_call(
      _msg2_body,
      grid=(grid,),
      in_specs=[
          pl.BlockSpec((blk, f1), lambda i: (i, 0)),
          pl.BlockSpec((blk, f1), lambda i: (i, 0)),
          pl.BlockSpec((blk, f1), lambda i: (i, 0)),
          dspec, dspec,
          pl.BlockSpec((1, f1), lambda i: (0, 0)),
          pl.BlockSpec((f1, f2), lambda i: (0, 0)),
      ],
      out_specs=pl.BlockSpec((blk, f2), lambda i: (i, 0)),
      out_shape=jax.ShapeDtypeStruct((n_pad, f2), jnp.float32),
  )(p[0], p[1], msg1, d0r, d1r, conv1_b.reshape(1, f1), conv2_W)

  q = _make_agg(n_pad, k, f2)(src_t, dst_t, msg2)

  out = pl.pallas_call(
      _final_body,
      grid=(1,),
      in_specs=[
          pl.BlockSpec((n_pad, f2), lambda i: (0, 0)),
          pl.BlockSpec((n_pad, f2), lambda i: (0, 0)),
          pl.BlockSpec((n_pad, f2), lambda i: (0, 0)),
          pl.BlockSpec((n_pad, 1), lambda i: (0, 0)),
          pl.BlockSpec((n_pad, 1), lambda i: (0, 0)),
          pl.BlockSpec((1, f2), lambda i: (0, 0)),
          pl.BlockSpec((1, n_pad), lambda i: (0, 0)),
          pl.BlockSpec(fc1_W.shape, lambda i: (0, 0)),
          pl.BlockSpec((1, fc1_b.shape[0]), lambda i: (0, 0)),
          pl.BlockSpec(fc2_W.shape, lambda i: (0, 0)),
          pl.BlockSpec((1, fc2_b.shape[0]), lambda i: (0, 0)),
      ],
      out_specs=pl.BlockSpec((G, 2 * LATENT), lambda i: (0, 0)),
      out_shape=jax.ShapeDtypeStruct((G, 2 * LATENT), jnp.float32),
  )(q[0], q[1], msg2, d0r, d1r, conv2_b.reshape(1, f2), batch_r,
    fc1_W, fc1_b.reshape(1, -1), fc2_W, fc2_b.reshape(1, -1))

  return (out[:, :LATENT], out[:, LATENT:])
